# baseline (speedup 1.0000x reference)
import jax
import jax.numpy as jnp
from jax import lax
from jax.experimental import pallas as pl
from jax.experimental.pallas import tpu as pltpu
from jax.experimental.pallas import tpu_sc as plsc

D = 64
DP = 128
KNEG = 20
NC, NS = 2, 16
NW = NC * NS
ROWS = 24
GSZ = 16
GN = GSZ * KNEG


def _relayout_body(inT, outT, in2, out2, slab_i, slab_o, tail_i, tail_o,
                   sem_i, sem_o):
    v = inT.shape[1]
    w = slab_i.shape[2]
    wr = w // 2
    nslab = v // w
    wid = lax.axis_index("s") * NC + lax.axis_index("c")
    iota = lax.iota(jnp.int32, 16)
    rowvecs = [iota + d0 for d0 in (0, 16, 32, 48)]

    for tb, (src, dst) in enumerate(((inT, in2), (outT, out2))):
        cnt = (nslab - wid + NW - 1) // NW

        def _fire_in(i, p, src=src):
            slab = wid + i * NW
            pltpu.async_copy(src.at[:, pl.ds(slab * w, w)],
                             slab_i.at[p], sem_i.at[p])

        _fire_in(0, 0)

        @pl.when(cnt >= 2)
        def _():
            _fire_in(1, 1)

        @pl.loop(0, cnt)
        def _(i):
            p = lax.rem(i, 2)
            slab = wid + i * NW
            pltpu.make_async_copy(src.at[:, pl.ds(0, w)],
                                  slab_i.at[p], sem_i.at[p]).wait()

            @pl.when(i >= 2)
            def _():
                pltpu.make_async_copy(slab_o.at[p],
                                      in2.at[pl.ds(0, wr), :],
                                      sem_o.at[p]).wait()

            @plsc.parallel_loop(0, wr, unroll=8)
            def _(r):
                for ci, c0 in enumerate(range(0, 128, 16)):
                    col = jnp.full((16,), 2 * r + (1 if c0 >= 64 else 0),
                                   jnp.int32)
                    val = plsc.load_gather(slab_i.at[p],
                                           [rowvecs[ci % 4], col])
                    slab_o[p, r, pl.ds(c0, 16)] = val
            pltpu.async_copy(slab_o.at[p],
                             dst.at[pl.ds(slab * wr, wr), :], sem_o.at[p])

            @pl.when(i < cnt - 2)
            def _():
                _fire_in(i + 2, lax.rem(i, 2))

        for p in range(2):
            @pl.when(cnt >= p + 1)
            def _():
                pltpu.make_async_copy(slab_o.at[p],
                                      in2.at[pl.ds(0, wr), :],
                                      sem_o.at[p]).wait()

        @pl.when(wid == 0)
        def _():
            ntail = v - nslab * w
            if ntail:
                pltpu.sync_copy(src.at[:, pl.ds(nslab * w, ntail)],
                                tail_i.at[tb])

                @plsc.parallel_loop(0, ntail // 2, unroll=8)
                def _(r):
                    for ci, c0 in enumerate(range(0, 128, 16)):
                        col = jnp.full((16,), 2 * r + (1 if c0 >= 64 else 0),
                                       jnp.int32)
                        val = plsc.load_gather(tail_i.at[tb],
                                               [rowvecs[ci % 4], col])
                        tail_o[tb, r, pl.ds(c0, 16)] = val
                pltpu.sync_copy(tail_o.at[tb],
                                dst.at[pl.ds(nslab * wr, ntail // 2), :])


def _fire_group(g, in2, out2, idx_c, idx_o, idx_n,
                vc_buf, vo_buf, vng_buf, sem):
    col0 = pl.multiple_of(g * GSZ, 8)
    nbase = pl.multiple_of(g * GN, 8)
    pltpu.async_copy(in2.at[idx_c.at[pl.ds(col0, GSZ)]], vc_buf, sem)
    pltpu.async_copy(out2.at[idx_o.at[pl.ds(col0, GSZ)]], vo_buf, sem)
    pltpu.async_copy(out2.at[idx_n.at[pl.ds(nbase, 128)]],
                     vng_buf.at[pl.ds(0, 128), :], sem)
    pltpu.async_copy(out2.at[idx_n.at[pl.ds(nbase + 128, 128)]],
                     vng_buf.at[pl.ds(128, 128), :], sem)
    pltpu.async_copy(out2.at[idx_n.at[pl.ds(nbase + 256, 64)]],
                     vng_buf.at[pl.ds(256, 64), :], sem)


def _wait_group(in2, vc_buf, vo_buf, vng_buf, sem):
    pltpu.make_async_copy(in2.at[pl.ds(0, GSZ)], vc_buf, sem).wait()
    pltpu.make_async_copy(in2.at[pl.ds(0, GSZ)], vo_buf, sem).wait()
    pltpu.make_async_copy(in2.at[pl.ds(0, 128)],
                          vng_buf.at[pl.ds(0, 128), :], sem).wait()
    pltpu.make_async_copy(in2.at[pl.ds(0, 128)],
                          vng_buf.at[pl.ds(128, 128), :], sem).wait()
    pltpu.make_async_copy(in2.at[pl.ds(0, 64)],
                          vng_buf.at[pl.ds(256, 64), :], sem).wait()


def _sc_scores_body(in2, out2, cen_hbm, ctx_hbm, neg_hbm, scores_hbm,
                    idx_c, idx_o, idx_n, par_c, par_o, par_n,
                    vc_g, vo_g, vng, scores_v, sem_a):
    bpw = idx_c.shape[0]
    ng = bpw // GSZ
    wid = lax.axis_index("s") * NC + lax.axis_index("c")
    base = wid * bpw

    pltpu.sync_copy(cen_hbm.at[pl.ds(base, bpw)], idx_c)
    pltpu.sync_copy(ctx_hbm.at[pl.ds(base, bpw)], idx_o)
    pltpu.sync_copy(neg_hbm.at[pl.ds(base * KNEG, bpw * KNEG)], idx_n)

    @pl.loop(0, bpw // 16)
    def _(i):
        off = pl.multiple_of(i * 16, 8)
        a = idx_c[pl.ds(off, 16)]
        idx_c[pl.ds(off, 16)] = lax.shift_right_logical(a, 1)
        par_c[pl.ds(off, 16)] = lax.shift_left(lax.bitwise_and(a, 1), 6)
        w = idx_o[pl.ds(off, 16)]
        idx_o[pl.ds(off, 16)] = lax.shift_right_logical(w, 1)
        par_o[pl.ds(off, 16)] = lax.shift_left(lax.bitwise_and(w, 1), 6)

    @pl.loop(0, bpw * KNEG // 16)
    def _(i):
        off = pl.multiple_of(i * 16, 8)
        a = idx_n[pl.ds(off, 16)]
        idx_n[pl.ds(off, 16)] = lax.shift_right_logical(a, 1)
        par_n[pl.ds(off, 16)] = lax.shift_left(lax.bitwise_and(a, 1), 6)

    _fire_group(0, in2, out2, idx_c, idx_o, idx_n,
                vc_g.at[0], vo_g.at[0], vng.at[0], sem_a.at[0])
    _fire_group(1, in2, out2, idx_c, idx_o, idx_n,
                vc_g.at[1], vo_g.at[1], vng.at[1], sem_a.at[1])

    iota = lax.iota(jnp.int32, 16)
    iota_k = iota * KNEG
    big = jnp.full((16,), 1e4, jnp.float32)

    def _compute_group(g, vc_buf, vo_buf, vng_buf):
        col0 = pl.multiple_of(g * GSZ, 8)
        pc = par_c[pl.ds(col0, 16)]
        po = par_o[pl.ds(col0, 16)]
        nbase = g * GN
        for k_lo, k_hi, with_pos in ((0, 6, True), (6, 13, False),
                                     (13, 20, False)):
            nacc = (k_hi - k_lo) + (1 if with_pos else 0)
            accs = [jnp.zeros((16,), jnp.float32)] * nacc
            pns = [plsc.load_gather(par_n, [iota_k + (nbase + k)])
                   for k in range(k_lo, k_hi)]
            for d in range(D):
                vcc = plsc.load_gather(vc_buf, [iota, pc + d])
                if with_pos:
                    voc = plsc.load_gather(vo_buf, [iota, po + d])
                    accs[0] = accs[0] + vcc * voc
                for j, k in enumerate(range(k_lo, k_hi)):
                    i = j + (1 if with_pos else 0)
                    vnc = plsc.load_gather(vng_buf,
                                           [iota_k + k, pns[j] + d])
                    accs[i] = accs[i] + vnc * vcc
            if with_pos:
                scores_v[0, pl.ds(col0, 16)] = accs[0]
            for j, k in enumerate(range(k_lo, k_hi)):
                i = j + (1 if with_pos else 0)
                scores_v[k + 1, pl.ds(col0, 16)] = -accs[i]
        for r in range(KNEG + 1, ROWS):
            scores_v[r, pl.ds(col0, 16)] = big

    @pl.loop(0, ng)
    def _(t):
        p = lax.rem(t, 2)
        vc_buf, vo_buf, vng_buf = vc_g.at[p], vo_g.at[p], vng.at[p]
        sem = sem_a.at[p]
        _wait_group(in2, vc_buf, vo_buf, vng_buf, sem)
        _compute_group(t, vc_buf, vo_buf, vng_buf)

        @pl.when(t < ng - 2)
        def _():
            _fire_group(t + 2, in2, out2, idx_c, idx_o, idx_n,
                        vc_buf, vo_buf, vng_buf, sem)

    pltpu.sync_copy(scores_v, scores_hbm.at[:, pl.ds(base, bpw)])


def _tc_loss_body(s_ref, o_ref):
    x = s_ref[...]
    ls = jnp.minimum(x, 0.0) - jnp.log1p(jnp.exp(-jnp.abs(x)))
    row = lax.broadcasted_iota(jnp.int32, x.shape, 0)
    ls = jnp.where(row < KNEG + 1, ls, 0.0)
    o_ref[0, 0] = -jnp.sum(ls) / s_ref.shape[1]


def kernel(center, context, negatives, in_embed, out_embed):
    b = center.shape[0]
    bpw = b // NW
    v = in_embed.shape[0]
    negflat = negatives.reshape(-1)

    mesh = plsc.VectorSubcoreMesh(core_axis_name="c", subcore_axis_name="s")
    params = pltpu.CompilerParams(
        needs_layout_passes=False, use_tc_tiling_on_sc=True)

    in2, out2 = pl.kernel(
        _relayout_body,
        out_type=(jax.ShapeDtypeStruct((v // 2, DP), jnp.float32),
                  jax.ShapeDtypeStruct((v // 2, DP), jnp.float32)),
        mesh=mesh,
        compiler_params=params,
        scratch_types=[
            pltpu.VMEM((2, D, 256), jnp.float32),
            pltpu.VMEM((2, 128, 128), jnp.float32),
            pltpu.VMEM((2, D, D), jnp.float32),
            pltpu.VMEM((2, D // 2, 128), jnp.float32),
            pltpu.SemaphoreType.DMA((2,)),
            pltpu.SemaphoreType.DMA((2,)),
        ],
    )(in_embed.T, out_embed.T)

    scores = pl.kernel(
        _sc_scores_body,
        out_type=jax.ShapeDtypeStruct((ROWS, b), jnp.float32),
        mesh=mesh,
        compiler_params=params,
        scratch_types=[
            pltpu.VMEM((bpw,), jnp.int32),
            pltpu.VMEM((bpw,), jnp.int32),
            pltpu.VMEM((bpw * KNEG,), jnp.int32),
            pltpu.VMEM((bpw,), jnp.int32),
            pltpu.VMEM((bpw,), jnp.int32),
            pltpu.VMEM((bpw * KNEG,), jnp.int32),
            pltpu.VMEM((2, GSZ, DP), jnp.float32),
            pltpu.VMEM((2, GSZ, DP), jnp.float32),
            pltpu.VMEM((2, GN, DP), jnp.float32),
            pltpu.VMEM((ROWS, bpw), jnp.float32),
            pltpu.SemaphoreType.DMA((2,)),
        ],
    )(in2, out2, center, context, negflat)

    loss = pl.pallas_call(
        _tc_loss_body,
        out_shape=jax.ShapeDtypeStruct((1, 1), jnp.float32),
        in_specs=[pl.BlockSpec((ROWS, b), lambda: (0, 0))],
        out_specs=pl.BlockSpec(memory_space=pltpu.SMEM),
    )(scores)
    return loss[0, 0]


# speedup vs baseline: 1.4191x; 1.4191x over previous
import jax
import jax.numpy as jnp
from jax import lax
from jax.experimental import pallas as pl
from jax.experimental.pallas import tpu as pltpu
from jax.experimental.pallas import tpu_sc as plsc

D = 64
DP = 128
KNEG = 20
NC, NS = 2, 16
NW = NC * NS
ROWS = 24
GSZ = 16
GN = GSZ * KNEG


def _fire_group(g, in2, out2, idx_c, idx_o, idx_n,
                vc_buf, vo_buf, vng_buf, sem):
    col0 = pl.multiple_of(g * GSZ, 8)
    nbase = pl.multiple_of(g * GN, 8)
    pltpu.async_copy(in2.at[idx_c.at[pl.ds(col0, GSZ)]], vc_buf, sem)
    pltpu.async_copy(out2.at[idx_o.at[pl.ds(col0, GSZ)]], vo_buf, sem)
    pltpu.async_copy(out2.at[idx_n.at[pl.ds(nbase, 128)]],
                     vng_buf.at[pl.ds(0, 128), :], sem)
    pltpu.async_copy(out2.at[idx_n.at[pl.ds(nbase + 128, 128)]],
                     vng_buf.at[pl.ds(128, 128), :], sem)
    pltpu.async_copy(out2.at[idx_n.at[pl.ds(nbase + 256, 64)]],
                     vng_buf.at[pl.ds(256, 64), :], sem)


def _wait_group(in2, vc_buf, vo_buf, vng_buf, sem):
    pltpu.make_async_copy(in2.at[pl.ds(0, GSZ)], vc_buf, sem).wait()
    pltpu.make_async_copy(in2.at[pl.ds(0, GSZ)], vo_buf, sem).wait()
    pltpu.make_async_copy(in2.at[pl.ds(0, 128)],
                          vng_buf.at[pl.ds(0, 128), :], sem).wait()
    pltpu.make_async_copy(in2.at[pl.ds(0, 128)],
                          vng_buf.at[pl.ds(128, 128), :], sem).wait()
    pltpu.make_async_copy(in2.at[pl.ds(0, 64)],
                          vng_buf.at[pl.ds(256, 64), :], sem).wait()


def _sc_scores_body(in2, out2, cen_hbm, ctx_hbm, neg_hbm, scores_hbm,
                    idx_c, idx_o, idx_n, vc_g, vo_g, vng, scores_v, sem_a):
    bpw = idx_c.shape[0]
    ng = bpw // GSZ
    wid = lax.axis_index("s") * NC + lax.axis_index("c")
    base = wid * bpw

    pltpu.sync_copy(cen_hbm.at[pl.ds(base, bpw)], idx_c)
    pltpu.sync_copy(ctx_hbm.at[pl.ds(base, bpw)], idx_o)
    pltpu.sync_copy(neg_hbm.at[pl.ds(base * KNEG, bpw * KNEG)], idx_n)

    _fire_group(0, in2, out2, idx_c, idx_o, idx_n,
                vc_g.at[0], vo_g.at[0], vng.at[0], sem_a.at[0])
    _fire_group(1, in2, out2, idx_c, idx_o, idx_n,
                vc_g.at[1], vo_g.at[1], vng.at[1], sem_a.at[1])

    iota = lax.iota(jnp.int32, 16)
    iota_k = iota * KNEG
    big = jnp.full((16,), 1e4, jnp.float32)

    def _compute_group(g, vc_buf, vo_buf, vng_buf):
        col0 = pl.multiple_of(g * GSZ, 8)
        for k_lo, k_hi, with_pos in ((0, 6, True), (6, 13, False),
                                     (13, 20, False)):
            nacc = (k_hi - k_lo) + (1 if with_pos else 0)
            accs = [jnp.zeros((16,), jnp.float32)] * nacc
            for d in range(D):
                dcol = jnp.full((16,), d, jnp.int32)
                vcc = plsc.load_gather(vc_buf, [iota, dcol])
                if with_pos:
                    voc = plsc.load_gather(vo_buf, [iota, dcol])
                    accs[0] = accs[0] + vcc * voc
                for j, k in enumerate(range(k_lo, k_hi)):
                    i = j + (1 if with_pos else 0)
                    vnc = plsc.load_gather(vng_buf, [iota_k + k, dcol])
                    accs[i] = accs[i] + vnc * vcc
            if with_pos:
                scores_v[0, pl.ds(col0, 16)] = accs[0]
            for j, k in enumerate(range(k_lo, k_hi)):
                i = j + (1 if with_pos else 0)
                scores_v[k + 1, pl.ds(col0, 16)] = -accs[i]
        for r in range(KNEG + 1, ROWS):
            scores_v[r, pl.ds(col0, 16)] = big

    @pl.loop(0, ng)
    def _(t):
        p = lax.rem(t, 2)
        vc_buf, vo_buf, vng_buf = vc_g.at[p], vo_g.at[p], vng.at[p]
        sem = sem_a.at[p]
        _wait_group(in2, vc_buf, vo_buf, vng_buf, sem)
        _compute_group(t, vc_buf, vo_buf, vng_buf)

        @pl.when(t < ng - 2)
        def _():
            _fire_group(t + 2, in2, out2, idx_c, idx_o, idx_n,
                        vc_buf, vo_buf, vng_buf, sem)

    pltpu.sync_copy(scores_v, scores_hbm.at[:, pl.ds(base, bpw)])


def _tc_loss_body(s_ref, o_ref):
    x = s_ref[...]
    ls = jnp.minimum(x, 0.0) - jnp.log1p(jnp.exp(-jnp.abs(x)))
    row = lax.broadcasted_iota(jnp.int32, x.shape, 0)
    ls = jnp.where(row < KNEG + 1, ls, 0.0)
    o_ref[0, 0] = -jnp.sum(ls) / s_ref.shape[1]


def kernel(center, context, negatives, in_embed, out_embed):
    b = center.shape[0]
    bpw = b // NW
    negflat = negatives.reshape(-1)
    in2 = jnp.pad(in_embed, ((0, 0), (0, DP - D)))
    out2 = jnp.pad(out_embed, ((0, 0), (0, DP - D)))

    scores = pl.kernel(
        _sc_scores_body,
        out_type=jax.ShapeDtypeStruct((ROWS, b), jnp.float32),
        mesh=plsc.VectorSubcoreMesh(core_axis_name="c", subcore_axis_name="s"),
        compiler_params=pltpu.CompilerParams(
            needs_layout_passes=False, use_tc_tiling_on_sc=True),
        scratch_types=[
            pltpu.VMEM((bpw,), jnp.int32),
            pltpu.VMEM((bpw,), jnp.int32),
            pltpu.VMEM((bpw * KNEG,), jnp.int32),
            pltpu.VMEM((2, GSZ, DP), jnp.float32),
            pltpu.VMEM((2, GSZ, DP), jnp.float32),
            pltpu.VMEM((2, GN, DP), jnp.float32),
            pltpu.VMEM((ROWS, bpw), jnp.float32),
            pltpu.SemaphoreType.DMA((2,)),
        ],
    )(in2, out2, center, context, negflat)

    loss = pl.pallas_call(
        _tc_loss_body,
        out_shape=jax.ShapeDtypeStruct((1, 1), jnp.float32),
        in_specs=[pl.BlockSpec((ROWS, b), lambda: (0, 0))],
        out_specs=pl.BlockSpec(memory_space=pltpu.SMEM),
    )(scores)
    return loss[0, 0]
